# baseline (speedup 1.0000x reference)
import functools

import jax
import jax.numpy as jnp
from jax import lax
from jax.experimental import pallas as pl
from jax.experimental.pallas import tpu as pltpu
from jax.experimental.pallas import tpu_sc as plsc

_BATCH = 16384
_D = 64
_H1 = 500
_H2 = 250

_NC = 2
_NS = 16
_NW = _NC * _NS
_BPW = _BATCH // _NW


@functools.cache
def _make_sc_gather():
    mesh = plsc.VectorSubcoreMesh(core_axis_name="c", subcore_axis_name="s")

    @functools.partial(
        pl.kernel,
        mesh=mesh,
        out_type=(
            jax.ShapeDtypeStruct((_BATCH, _D), jnp.float32),
            jax.ShapeDtypeStruct((_BATCH, _D), jnp.float32),
        ),
        scratch_types=[
            pltpu.VMEM((_BPW,), jnp.int32),
            pltpu.VMEM((_BPW, _D), jnp.float32),
            pltpu.VMEM((_BPW,), jnp.int32),
            pltpu.VMEM((_BPW, _D), jnp.float32),
            pltpu.SemaphoreType.DMA,
            pltpu.SemaphoreType.DMA,
        ],
        compiler_params=pltpu.CompilerParams(use_tc_tiling_on_sc=False),
    )
    def gather_kernel(uid_hbm, mid_hbm, utab_hbm, mtab_hbm, uf_out, mf_out,
                      uidx_v, urows_v, midx_v, mrows_v, sem_u, sem_m):
        wid = lax.axis_index("s") * _NC + lax.axis_index("c")
        base = wid * _BPW
        pltpu.sync_copy(uid_hbm.at[pl.ds(base, _BPW)], uidx_v)
        pltpu.sync_copy(mid_hbm.at[pl.ds(base, _BPW)], midx_v)
        cu = pltpu.async_copy(utab_hbm.at[uidx_v], urows_v, sem_u)
        cm = pltpu.async_copy(mtab_hbm.at[midx_v], mrows_v, sem_m)
        cu.wait()
        cm.wait()
        pltpu.sync_copy(urows_v, uf_out.at[pl.ds(base, _BPW)])
        pltpu.sync_copy(mrows_v, mf_out.at[pl.ds(base, _BPW)])

    return gather_kernel


_BB = 2048


def _mlp_block(uf_ref, mf_ref, w1u_ref, w1m_ref, b1_ref, g1_ref, be1_ref,
               w2_ref, b2_ref, g2_ref, be2_ref, w3_ref, b3_ref, out_ref):
    h = (jnp.dot(uf_ref[...], w1u_ref[...], preferred_element_type=jnp.float32)
         + jnp.dot(mf_ref[...], w1m_ref[...], preferred_element_type=jnp.float32)
         + b1_ref[...])
    mu = jnp.mean(h, axis=-1, keepdims=True)
    var = jnp.mean((h - mu) ** 2, axis=-1, keepdims=True)
    h = (h - mu) * lax.rsqrt(var + 1e-5) * g1_ref[...] + be1_ref[...]
    h = jnp.maximum(h, 0.0)
    h = jnp.dot(h, w2_ref[...], preferred_element_type=jnp.float32) + b2_ref[...]
    mu = jnp.mean(h, axis=-1, keepdims=True)
    var = jnp.mean((h - mu) ** 2, axis=-1, keepdims=True)
    h = (h - mu) * lax.rsqrt(var + 1e-5) * g2_ref[...] + be2_ref[...]
    h = jnp.maximum(h, 0.0)
    o = jnp.sum(h * w3_ref[...], axis=-1, keepdims=True) + b3_ref[...]
    out_ref[...] = jax.nn.sigmoid(o) * 5.5


def _mlp(uf, mf, w1u, w1m, b1, g1, be1, w2, b2, g2, be2, w3t, b3):
    grid = _BATCH // _BB
    full = lambda shape: pl.BlockSpec(shape, lambda i: (0,) * len(shape))
    return pl.pallas_call(
        _mlp_block,
        grid=(grid,),
        in_specs=[
            pl.BlockSpec((_BB, _D), lambda i: (i, 0)),
            pl.BlockSpec((_BB, _D), lambda i: (i, 0)),
            full((_D, _H1)),
            full((_D, _H1)),
            full((1, _H1)),
            full((1, _H1)),
            full((1, _H1)),
            full((_H1, _H2)),
            full((1, _H2)),
            full((1, _H2)),
            full((1, _H2)),
            full((1, _H2)),
            full((1, 1)),
        ],
        out_specs=pl.BlockSpec((_BB, 1), lambda i: (i, 0)),
        out_shape=jax.ShapeDtypeStruct((_BATCH, 1), jnp.float32),
    )(uf, mf, w1u, w1m, b1, g1, be1, w2, b2, g2, be2, w3t, b3)


def kernel(user_ids, movie_ids, user_table, movie_table,
           W1, b1, g1, be1, W2, b2, g2, be2, W3, b3):
    uf, mf = _make_sc_gather()(user_ids.astype(jnp.int32),
                               movie_ids.astype(jnp.int32),
                               user_table, movie_table)
    out = _mlp(
        uf, mf,
        W1[:_D], W1[_D:],
        b1.reshape(1, _H1), g1.reshape(1, _H1), be1.reshape(1, _H1),
        W2,
        b2.reshape(1, _H2), g2.reshape(1, _H2), be2.reshape(1, _H2),
        W3.reshape(1, _H2), b3.reshape(1, 1),
    )
    return out


# speedup vs baseline: 1.5873x; 1.5873x over previous
import functools

import jax
import jax.numpy as jnp
from jax import lax
from jax.experimental import pallas as pl
from jax.experimental.pallas import tpu as pltpu
from jax.experimental.pallas import tpu_sc as plsc

_BATCH = 16384
_D = 64
_H1 = 500
_H2 = 250
_N = 1000000

_PW = 2048
_NPACK = (_N + _PW - 1) // _PW
_QROWS = _NPACK * (_PW // 2)

_NC = 2
_NS = 16
_NW = _NC * _NS
_BPW = _BATCH // _NW
_CHUNK = 256


def _pack_block(tu_ref, tm_ref, ou_ref, om_ref):
    for t_ref, o_ref in ((tu_ref, ou_ref), (tm_ref, om_ref)):
        tr = jnp.transpose(t_ref[...], (1, 0))
        half = _PW // 2
        o_ref[...] = jnp.concatenate([tr[:half], tr[half:]], axis=1)


def _pack(ttu, ttm):
    return pl.pallas_call(
        _pack_block,
        grid=(_NPACK,),
        in_specs=[
            pl.BlockSpec((_D, _PW), lambda i: (0, i)),
            pl.BlockSpec((_D, _PW), lambda i: (0, i)),
        ],
        out_specs=[
            pl.BlockSpec((_PW // 2, 2 * _D), lambda i: (i, 0)),
            pl.BlockSpec((_PW // 2, 2 * _D), lambda i: (i, 0)),
        ],
        out_shape=[
            jax.ShapeDtypeStruct((_QROWS, 2 * _D), jnp.float32),
            jax.ShapeDtypeStruct((_QROWS, 2 * _D), jnp.float32),
        ],
    )(ttu, ttm)


@functools.cache
def _make_sc_gather():
    mesh = plsc.VectorSubcoreMesh(core_axis_name="c", subcore_axis_name="s")

    @functools.partial(
        pl.kernel,
        mesh=mesh,
        out_type=(
            jax.ShapeDtypeStruct((_BATCH, 2 * _D), jnp.float32),
            jax.ShapeDtypeStruct((_BATCH, 2 * _D), jnp.float32),
        ),
        scratch_types=[
            pltpu.VMEM((_BPW,), jnp.int32),
            pltpu.VMEM((_BPW,), jnp.int32),
            pltpu.VMEM((_CHUNK,), jnp.int32),
            pltpu.VMEM((_CHUNK,), jnp.int32),
            pltpu.VMEM((_CHUNK, 2 * _D), jnp.float32),
            pltpu.VMEM((_CHUNK, 2 * _D), jnp.float32),
            pltpu.SemaphoreType.DMA,
            pltpu.SemaphoreType.DMA,
        ],
    )
    def gather_kernel(uid_hbm, mid_hbm, up_hbm, mp_hbm, xu_out, xm_out,
                      uid_v, mid_v, qu_v, qm_v, du_v, dm_v, sem_u, sem_m):
        wid = lax.axis_index("s") * _NC + lax.axis_index("c")
        base = wid * _BPW
        pltpu.sync_copy(uid_hbm.at[pl.ds(base, _BPW)], uid_v)
        pltpu.sync_copy(mid_hbm.at[pl.ds(base, _BPW)], mid_v)
        for ch in range(_BPW // _CHUNK):
            off = ch * _CHUNK
            for g in range(_CHUNK // 16):
                u = uid_v[pl.ds(off + g * 16, 16)]
                m = mid_v[pl.ds(off + g * 16, 16)]
                qu_v[pl.ds(g * 16, 16)] = (
                    ((u >> 11) << 10) | (u & (_PW // 2 - 1)))
                qm_v[pl.ds(g * 16, 16)] = (
                    ((m >> 11) << 10) | (m & (_PW // 2 - 1)))
            cu = pltpu.async_copy(up_hbm.at[qu_v], du_v, sem_u)
            cm = pltpu.async_copy(mp_hbm.at[qm_v], dm_v, sem_m)
            cu.wait()
            cm.wait()
            pltpu.sync_copy(du_v, xu_out.at[pl.ds(base + off, _CHUNK)])
            pltpu.sync_copy(dm_v, xm_out.at[pl.ds(base + off, _CHUNK)])

    return gather_kernel


_BB = 2048


def _mlp_block(xu_ref, xm_ref, hu_ref, hm_ref, w1_ref, b1_ref, g1_ref,
               be1_ref, w2_ref, b2_ref, g2_ref, be2_ref, w3_ref, b3_ref,
               out_ref):
    uf = jnp.where(hu_ref[...][:, :1] > 0.5,
                   xu_ref[...][:, _D:], xu_ref[...][:, :_D])
    mf = jnp.where(hm_ref[...][:, :1] > 0.5,
                   xm_ref[...][:, _D:], xm_ref[...][:, :_D])
    x = jnp.concatenate([uf, mf], axis=1)
    h = jnp.dot(x, w1_ref[...], preferred_element_type=jnp.float32) + b1_ref[...]
    mu = jnp.mean(h, axis=-1, keepdims=True)
    var = jnp.mean((h - mu) ** 2, axis=-1, keepdims=True)
    h = (h - mu) * lax.rsqrt(var + 1e-5) * g1_ref[...] + be1_ref[...]
    h = jnp.maximum(h, 0.0)
    h = jnp.dot(h, w2_ref[...], preferred_element_type=jnp.float32) + b2_ref[...]
    mu = jnp.mean(h, axis=-1, keepdims=True)
    var = jnp.mean((h - mu) ** 2, axis=-1, keepdims=True)
    h = (h - mu) * lax.rsqrt(var + 1e-5) * g2_ref[...] + be2_ref[...]
    h = jnp.maximum(h, 0.0)
    o = jnp.sum(h * w3_ref[...], axis=-1, keepdims=True) + b3_ref[...]
    out_ref[...] = jax.nn.sigmoid(o) * 5.5


def _mlp(xu, xm, hu8, hm8, w1, b1, g1, be1, w2, b2, g2, be2, w3t, b3):
    grid = _BATCH // _BB
    full = lambda shape: pl.BlockSpec(shape, lambda i: (0,) * len(shape))
    return pl.pallas_call(
        _mlp_block,
        grid=(grid,),
        in_specs=[
            pl.BlockSpec((_BB, 2 * _D), lambda i: (i, 0)),
            pl.BlockSpec((_BB, 2 * _D), lambda i: (i, 0)),
            pl.BlockSpec((_BB, 8), lambda i: (i, 0)),
            pl.BlockSpec((_BB, 8), lambda i: (i, 0)),
            full((2 * _D, _H1)),
            full((1, _H1)),
            full((1, _H1)),
            full((1, _H1)),
            full((_H1, _H2)),
            full((1, _H2)),
            full((1, _H2)),
            full((1, _H2)),
            full((1, _H2)),
            full((1, 1)),
        ],
        out_specs=pl.BlockSpec((_BB, 1), lambda i: (i, 0)),
        out_shape=jax.ShapeDtypeStruct((_BATCH, 1), jnp.float32),
    )(xu, xm, hu8, hm8, w1, b1, g1, be1, w2, b2, g2, be2, w3t, b3)


def kernel(user_ids, movie_ids, user_table, movie_table,
           W1, b1, g1, be1, W2, b2, g2, be2, W3, b3):
    uid = user_ids.astype(jnp.int32)
    mid = movie_ids.astype(jnp.int32)
    upack, mpack = _pack(user_table.T, movie_table.T)
    xu, xm = _make_sc_gather()(uid, mid, upack, mpack)
    hu8 = jnp.broadcast_to(
        ((uid >> 10) & 1).astype(jnp.float32)[:, None], (_BATCH, 8))
    hm8 = jnp.broadcast_to(
        ((mid >> 10) & 1).astype(jnp.float32)[:, None], (_BATCH, 8))
    out = _mlp(
        xu, xm, hu8, hm8, W1,
        b1.reshape(1, _H1), g1.reshape(1, _H1), be1.reshape(1, _H1),
        W2,
        b2.reshape(1, _H2), g2.reshape(1, _H2), be2.reshape(1, _H2),
        W3.reshape(1, _H2), b3.reshape(1, 1),
    )
    return out
